# baseline (device time: 31619 ns/iter reference)
import jax
import jax.numpy as jnp
from jax import lax
from jax.experimental import pallas as pl
from jax.experimental.pallas import tpu as pltpu

N_DEV = 8
B = 2
SQ = 256
HQ = 8
DH = 64
BH = B * HQ
SCALE = 0.125
PACK = 128
REGIONS = ((0, 2), (2, 2), (4, 1), (5, 1), (6, 1), (7, 1))
DSEQ = (
    (1, 2, 4), (4, 2, 1),
    (1, 2, 4), (4, 2, 1), (1, 2, 4), (4, 2, 1),
    (2, 1, 4),
)
N_CHAINS = 7
NROWS = 8


def kernel(x, Wq, Wo, K_ext, V_ext):
    def body(x_ref, wq_ref, wo_ref, k_ref, v_ref, out_ref,
             comm_ref, l_comm_ref, send_sems, recv_sems):
        my_pos = lax.axis_index("i")

        barrier_sem = pltpu.get_barrier_semaphore()
        for d in (1, 2, 4):
            pl.semaphore_signal(
                barrier_sem, inc=1,
                device_id=(jnp.bitwise_xor(my_pos, d),),
                device_id_type=pl.DeviceIdType.MESH,
            )
        pl.semaphore_wait(barrier_sem, 3)

        def attn_heads(b, q_b, h0, h1):
            os, ls = [], []
            for h in range(h0, h1):
                q_bh = q_b[:, h * DH:(h + 1) * DH]
                k_bh = k_ref[b, :, h, :]
                v_bh = v_ref[b, :, h, :]
                s = lax.dot_general(
                    q_bh, k_bh, (((1,), (1,)), ((), ())),
                    preferred_element_type=jnp.float32,
                ) * SCALE
                p = jnp.exp(s)
                os.append(jnp.dot(p, v_bh,
                                  preferred_element_type=jnp.float32))
                ls.append(jnp.sum(p, axis=1, keepdims=True))
            return os, ls

        def pack_o(row, os):
            for j in range(len(os) // 2):
                comm_ref[0, row + j] = jnp.concatenate(
                    [os[2 * j], os[2 * j + 1]], axis=1).astype(jnp.bfloat16)

        def start_step(r, s):
            sem = 3 * r + s
            if r < len(REGIONS):
                row0, nrows = REGIONS[r]
                rows = pl.ds(row0, nrows)
                src, dst = comm_ref.at[0, rows], comm_ref.at[1 + s, rows]
            else:
                src, dst = l_comm_ref.at[0], l_comm_ref.at[1 + s]
            rdma = pltpu.make_async_remote_copy(
                src_ref=src,
                dst_ref=dst,
                send_sem=send_sems.at[sem],
                recv_sem=recv_sems.at[sem],
                device_id=(jnp.bitwise_xor(my_pos, DSEQ[r][s]),),
                device_id_type=pl.DeviceIdType.MESH,
            )
            rdma.start()
            return rdma

        def merge(r, s):
            if r < len(REGIONS):
                row0, nrows = REGIONS[r]
                comm_ref[0, row0:row0 + nrows] = (
                    comm_ref[0, row0:row0 + nrows]
                    + comm_ref[1 + s, row0:row0 + nrows])
            else:
                l_comm_ref[0] = l_comm_ref[0] + l_comm_ref[1 + s]

        pend = [None] * N_CHAINS
        q0 = jnp.dot(x_ref[0], wq_ref[...],
                     preferred_element_type=jnp.float32)
        os_a, ls_a = attn_heads(0, q0, 0, HQ // 2)
        pack_o(0, os_a)
        pend[0] = start_step(0, 0)
        os_b, ls_b = attn_heads(0, q0, HQ // 2, HQ)
        pack_o(2, os_b)
        pend[1] = start_step(1, 0)
        q1 = jnp.dot(x_ref[1], wq_ref[...],
                     preferred_element_type=jnp.float32)
        ls1 = []
        for k in range(4):
            os_k, ls_k = attn_heads(1, q1, 2 * k, 2 * k + 2)
            pack_o(4 + k, os_k)
            pend[2 + k] = start_step(2 + k, 0)
            ls1 += ls_k
        l_comm_ref[0] = jnp.concatenate(
            ls_a + ls_b + ls1, axis=1).astype(jnp.bfloat16)
        pend[6] = start_step(6, 0)

        for s in range(2):
            for r in range(N_CHAINS):
                pend[r].wait()
                merge(r, s)
                pend[r] = start_step(r, s + 1)

        def half_fin(b, half):
            cols = []
            for h in range(4 * half, 4 * half + 4):
                off = (h % 2) * DH
                o = comm_ref[0, 2 * (2 * b + half) + h % 4 // 2, :,
                             off:off + DH].astype(jnp.float32)
                l = l_comm_ref[0][:, b * HQ + h:b * HQ + h + 1].astype(
                    jnp.float32)
                cols.append(o / l)
            attn_half = jnp.concatenate(cols, axis=1)
            return jnp.dot(attn_half, wo_ref[4 * half * DH:
                                             (4 * half + 4) * DH, :],
                           preferred_element_type=jnp.float32)

        pend[6].wait()
        merge(6, 2)
        pend[0].wait()
        merge(0, 2)
        pend[1].wait()
        merge(1, 2)
        out_ref[0] = half_fin(0, 0) + half_fin(0, 1)
        pend[2].wait()
        merge(2, 2)
        pend[3].wait()
        merge(3, 2)
        part1 = half_fin(1, 0)
        pend[4].wait()
        merge(4, 2)
        pend[5].wait()
        merge(5, 2)
        out_ref[1] = part1 + half_fin(1, 1)

    return pl.pallas_call(
        body,
        out_shape=jax.ShapeDtypeStruct((B, SQ, 768), jnp.float32),
        in_specs=[
            pl.BlockSpec(memory_space=pltpu.VMEM),
            pl.BlockSpec(memory_space=pltpu.VMEM),
            pl.BlockSpec(memory_space=pltpu.VMEM),
            pl.BlockSpec(memory_space=pltpu.VMEM),
            pl.BlockSpec(memory_space=pltpu.VMEM),
        ],
        out_specs=pl.BlockSpec(memory_space=pltpu.VMEM),
        scratch_shapes=[
            pltpu.VMEM((4, NROWS, SQ, PACK), jnp.bfloat16),
            pltpu.VMEM((4, SQ, BH), jnp.bfloat16),
            pltpu.SemaphoreType.DMA((21,)),
            pltpu.SemaphoreType.DMA((21,)),
        ],
        compiler_params=pltpu.CompilerParams(collective_id=0),
    )(x, Wq, Wo, K_ext, V_ext)


# device time: 30364 ns/iter; 1.0413x vs baseline; 1.0413x over previous
import jax
import jax.numpy as jnp
from jax import lax
from jax.experimental import pallas as pl
from jax.experimental.pallas import tpu as pltpu

N_DEV = 8
B = 2
SQ = 256
HQ = 8
DH = 64
BH = B * HQ
SCALE = 0.125
PACK = 128
REGIONS = ((0, 2), (2, 2), (4, 1), (5, 1), (6, 1), (7, 1))
DSEQ = (
    (1, 3, 4), (4, 3, 1),
    (1, 3, 4), (4, 3, 1), (3, 1, 4), (4, 1, 3),
    (3, 4, 1),
)
BARRIER_PARTNERS = (1, 3, 4)
N_CHAINS = 7
NROWS = 8


def kernel(x, Wq, Wo, K_ext, V_ext):
    def body(x_ref, wq_ref, wo_ref, k_ref, v_ref, out_ref,
             comm_ref, l_comm_ref, send_sems, recv_sems):
        my_pos = lax.axis_index("i")

        barrier_sem = pltpu.get_barrier_semaphore()
        for d in BARRIER_PARTNERS:
            pl.semaphore_signal(
                barrier_sem, inc=1,
                device_id=(jnp.bitwise_xor(my_pos, d),),
                device_id_type=pl.DeviceIdType.MESH,
            )
        pl.semaphore_wait(barrier_sem, 3)

        def attn_heads(b, q_b, h0, h1):
            os, ls = [], []
            for h in range(h0, h1):
                q_bh = q_b[:, h * DH:(h + 1) * DH]
                k_bh = k_ref[b, :, h, :]
                v_bh = v_ref[b, :, h, :]
                s = lax.dot_general(
                    q_bh, k_bh, (((1,), (1,)), ((), ())),
                    preferred_element_type=jnp.float32,
                ) * SCALE
                p = jnp.exp(s)
                os.append(jnp.dot(p, v_bh,
                                  preferred_element_type=jnp.float32))
                ls.append(jnp.sum(p, axis=1, keepdims=True))
            return os, ls

        def pack_o(row, os):
            for j in range(len(os) // 2):
                comm_ref[0, row + j] = jnp.concatenate(
                    [os[2 * j], os[2 * j + 1]], axis=1).astype(jnp.bfloat16)

        def start_step(r, s):
            sem = 3 * r + s
            if r < len(REGIONS):
                row0, nrows = REGIONS[r]
                rows = pl.ds(row0, nrows)
                src, dst = comm_ref.at[0, rows], comm_ref.at[1 + s, rows]
            else:
                src, dst = l_comm_ref.at[0], l_comm_ref.at[1 + s]
            rdma = pltpu.make_async_remote_copy(
                src_ref=src,
                dst_ref=dst,
                send_sem=send_sems.at[sem],
                recv_sem=recv_sems.at[sem],
                device_id=(jnp.bitwise_xor(my_pos, DSEQ[r][s]),),
                device_id_type=pl.DeviceIdType.MESH,
            )
            rdma.start()
            return rdma

        def merge(r, s):
            if r < len(REGIONS):
                row0, nrows = REGIONS[r]
                comm_ref[0, row0:row0 + nrows] = (
                    comm_ref[0, row0:row0 + nrows]
                    + comm_ref[1 + s, row0:row0 + nrows])
            else:
                l_comm_ref[0] = l_comm_ref[0] + l_comm_ref[1 + s]

        pend = [None] * N_CHAINS
        q0 = jnp.dot(x_ref[0], wq_ref[...],
                     preferred_element_type=jnp.float32)
        os_a, ls_a = attn_heads(0, q0, 0, HQ // 2)
        pack_o(0, os_a)
        pend[0] = start_step(0, 0)
        os_b, ls_b = attn_heads(0, q0, HQ // 2, HQ)
        pack_o(2, os_b)
        pend[1] = start_step(1, 0)
        q1 = jnp.dot(x_ref[1], wq_ref[...],
                     preferred_element_type=jnp.float32)
        ls1 = []
        for k in range(4):
            os_k, ls_k = attn_heads(1, q1, 2 * k, 2 * k + 2)
            pack_o(4 + k, os_k)
            pend[2 + k] = start_step(2 + k, 0)
            ls1 += ls_k
        l_comm_ref[0] = jnp.concatenate(
            ls_a + ls_b + ls1, axis=1).astype(jnp.bfloat16)
        pend[6] = start_step(6, 0)

        for s in range(2):
            for r in range(N_CHAINS):
                pend[r].wait()
                merge(r, s)
                pend[r] = start_step(r, s + 1)

        def half_fin(b, half):
            cols = []
            for h in range(4 * half, 4 * half + 4):
                off = (h % 2) * DH
                o = comm_ref[0, 2 * (2 * b + half) + h % 4 // 2, :,
                             off:off + DH].astype(jnp.float32)
                l = l_comm_ref[0][:, b * HQ + h:b * HQ + h + 1].astype(
                    jnp.float32)
                cols.append(o / l)
            attn_half = jnp.concatenate(cols, axis=1)
            return jnp.dot(attn_half, wo_ref[4 * half * DH:
                                             (4 * half + 4) * DH, :],
                           preferred_element_type=jnp.float32)

        pend[6].wait()
        merge(6, 2)
        pend[0].wait()
        merge(0, 2)
        pend[1].wait()
        merge(1, 2)
        out_ref[0] = half_fin(0, 0) + half_fin(0, 1)
        pend[2].wait()
        merge(2, 2)
        pend[3].wait()
        merge(3, 2)
        part1 = half_fin(1, 0)
        pend[4].wait()
        merge(4, 2)
        pend[5].wait()
        merge(5, 2)
        out_ref[1] = part1 + half_fin(1, 1)

    return pl.pallas_call(
        body,
        out_shape=jax.ShapeDtypeStruct((B, SQ, 768), jnp.float32),
        in_specs=[
            pl.BlockSpec(memory_space=pltpu.VMEM),
            pl.BlockSpec(memory_space=pltpu.VMEM),
            pl.BlockSpec(memory_space=pltpu.VMEM),
            pl.BlockSpec(memory_space=pltpu.VMEM),
            pl.BlockSpec(memory_space=pltpu.VMEM),
        ],
        out_specs=pl.BlockSpec(memory_space=pltpu.VMEM),
        scratch_shapes=[
            pltpu.VMEM((4, NROWS, SQ, PACK), jnp.bfloat16),
            pltpu.VMEM((4, SQ, BH), jnp.bfloat16),
            pltpu.SemaphoreType.DMA((21,)),
            pltpu.SemaphoreType.DMA((21,)),
        ],
        compiler_params=pltpu.CompilerParams(collective_id=0),
    )(x, Wq, Wo, K_ext, V_ext)
